# baseline (speedup 1.0000x reference)
import jax
import jax.numpy as jnp
from jax.experimental import pallas as pl
from jax.experimental.pallas import tpu as pltpu

N = 10000
NFEAT = 128
NHID = 64
NCLASS = 16

BM = 200
NM = N // BM

_DOT = (((1,), (0,)), ((), ()))


def _dot(a, b):
    return jax.lax.dot_general(a, b, _DOT, preferred_element_type=jnp.float32)


def _attention(ol, oh, om, avl_ref, avh_ref, avm_ref, av_ref):
    sl = jax.nn.sigmoid(_dot(ol, avl_ref[...]))
    sh = jax.nn.sigmoid(_dot(oh, avh_ref[...]))
    sm = jax.nn.sigmoid(_dot(om, avm_ref[...]))
    logits = [
        (sl * av_ref[0, j] + sh * av_ref[1, j] + sm * av_ref[2, j]) * (1.0 / 3.0)
        for j in range(3)
    ]
    mx = jnp.maximum(jnp.maximum(logits[0], logits[1]), logits[2])
    e0 = jnp.exp(logits[0] - mx)
    e1 = jnp.exp(logits[1] - mx)
    e2 = jnp.exp(logits[2] - mx)
    inv = 1.0 / (e0 + e1 + e2)
    return e0 * inv, e1 * inv, e2 * inv


def _fused_kernel(adjl_ref, adjh_ref, x_ref, wl_ref, wh_ref, wm_ref,
                  wl2_ref, wh2_ref, wm2_ref, avl_ref, avh_ref, avm_ref,
                  av_ref, avl2_ref, avh2_ref, avm2_ref, av2_ref, out_ref,
                  xp_s, hp_s):
    layer = pl.program_id(0)
    m = pl.program_id(1)
    rows = pl.ds(m * BM, BM)

    @pl.when(jnp.logical_and(layer == 0, m == 0))
    def _():
        xf = x_ref[...]
        xp_s[:, 0:NHID] = _dot(xf, wl_ref[...])
        xp_s[:, NHID:2 * NHID] = _dot(xf, wh_ref[...])

    @pl.when(layer == 0)
    def _():
        ol = jnp.maximum(_dot(adjl_ref[...], xp_s[:, 0:NHID]), 0.0)
        oh = jnp.maximum(_dot(adjh_ref[...], xp_s[:, NHID:2 * NHID]), 0.0)
        om = jnp.maximum(_dot(x_ref[rows, :], wm_ref[...]), 0.0)
        al, ah, am = _attention(ol, oh, om, avl_ref, avh_ref, avm_ref, av_ref)
        h = 3.0 * (al * ol + ah * oh + am * om)
        hp_s[rows, 0:NCLASS] = _dot(h, wl2_ref[...])
        hp_s[rows, NCLASS:2 * NCLASS] = _dot(h, wh2_ref[...])
        hp_s[rows, 2 * NCLASS:3 * NCLASS] = jnp.maximum(
            _dot(h, wm2_ref[...]), 0.0)

    @pl.when(layer == 1)
    def _():
        ol = jnp.maximum(_dot(adjl_ref[...], hp_s[:, 0:NCLASS]), 0.0)
        oh = jnp.maximum(_dot(adjh_ref[...], hp_s[:, NCLASS:2 * NCLASS]), 0.0)
        om = hp_s[rows, 2 * NCLASS:3 * NCLASS]
        al, ah, am = _attention(ol, oh, om, avl2_ref, avh2_ref, avm2_ref, av2_ref)
        o = 3.0 * (al * ol + ah * oh + am * om)
        z = o - jnp.max(o, axis=1, keepdims=True)
        out_ref[...] = z - jnp.log(jnp.sum(jnp.exp(z), axis=1, keepdims=True))


def kernel(x, adj_low, adj_high, weight_low, weight_high, weight_mlp,
           att_vec_low, att_vec_high, att_vec_mlp, att_vec, weight_low2,
           weight_high2, weight_mlp2, att_vec_low2, att_vec_high2,
           att_vec_mlp2, att_vec2):
    adj_spec = pl.BlockSpec((BM, N), lambda l, m: (m, 0))
    full_spec = lambda a, b: pl.BlockSpec((a, b), lambda l, m: (0, 0))

    out = pl.pallas_call(
        _fused_kernel,
        grid=(2, NM),
        in_specs=[
            adj_spec,
            adj_spec,
            full_spec(N, NFEAT),
            full_spec(NFEAT, NHID),
            full_spec(NFEAT, NHID),
            full_spec(NFEAT, NHID),
            full_spec(NHID, NCLASS),
            full_spec(NHID, NCLASS),
            full_spec(NHID, NCLASS),
            full_spec(NHID, 1),
            full_spec(NHID, 1),
            full_spec(NHID, 1),
            full_spec(3, 3),
            full_spec(NCLASS, 1),
            full_spec(NCLASS, 1),
            full_spec(NCLASS, 1),
            full_spec(3, 3),
        ],
        out_specs=pl.BlockSpec((BM, NCLASS), lambda l, m: (m, 0)),
        out_shape=jax.ShapeDtypeStruct((N, NCLASS), jnp.float32),
        scratch_shapes=[
            pltpu.VMEM((N, 2 * NHID), jnp.float32),
            pltpu.VMEM((N, 3 * NCLASS), jnp.float32),
        ],
        compiler_params=pltpu.CompilerParams(
            dimension_semantics=("arbitrary", "arbitrary"),
            vmem_limit_bytes=100 * 1024 * 1024),
    )(adj_low, adj_high, x, weight_low, weight_high, weight_mlp,
      weight_low2, weight_high2, weight_mlp2, att_vec_low, att_vec_high,
      att_vec_mlp, att_vec, att_vec_low2, att_vec_high2, att_vec_mlp2,
      att_vec2)

    return out
